# baseline (speedup 1.0000x reference)
import functools

import jax
import jax.numpy as jnp
from jax import lax
from jax.experimental import pallas as pl
from jax.experimental.pallas import tpu as pltpu
from jax.experimental.pallas import tpu_sc as plsc

_HID = 128
_VOCAB = 32
_NCORES = 2
_NSUB = 16
_NW = _NCORES * _NSUB
_C = 128
_G = 2
_MC = _G * _C
_NBUF = 2
_LEAD = 1


def _sc_embed(ids2d, table):
    n_rows = ids2d.shape[0] * _C
    b_per_w = n_rows // _NW
    macros = b_per_w // _MC
    quads = macros // _NBUF
    mesh = plsc.VectorSubcoreMesh(core_axis_name="c", subcore_axis_name="s")

    @functools.partial(
        pl.kernel,
        mesh=mesh,
        out_type=jax.ShapeDtypeStruct((n_rows, _HID), jnp.float32),
        scratch_types=[
            pltpu.VMEM((_NBUF, _G, _C), jnp.int32),
            pltpu.VMEM((_NBUF, _MC, _HID), jnp.float32),
            pltpu.VMEM_SHARED((_VOCAB, _HID), jnp.float32),
        ] + [pltpu.SemaphoreType.DMA] * (2 * _NBUF),
    )
    def run(ids_hbm, table_hbm, out_hbm, idx_v, rows_v, table_s, *sems):
        sg = sems[:_NBUF]
        so = sems[_NBUF:]
        wid = lax.axis_index("s") * _NCORES + lax.axis_index("c")
        row0 = wid * b_per_w
        irow0 = row0 // _C

        @pl.when(lax.axis_index("s") == 0)
        def _():
            pltpu.sync_copy(table_hbm, table_s)

        plsc.subcore_barrier()

        def load_ids(b, m):
            for k in range(_G):
                pltpu.sync_copy(
                    ids_hbm.at[irow0 + m * _G + k], idx_v.at[b, k])

        def fire_gather(b):
            for k in range(_G):
                pltpu.async_copy(
                    table_s.at[idx_v.at[b, k]],
                    rows_v.at[b, pl.ds(k * _C, _C)], sg[b])

        def wait_gather(b):
            for k in range(_G):
                pltpu.make_async_copy(
                    table_s.at[idx_v.at[b, k]],
                    rows_v.at[b, pl.ds(k * _C, _C)], sg[b]).wait()

        def fire_out(b, m):
            pltpu.async_copy(
                rows_v.at[b], out_hbm.at[pl.ds((irow0 + m * _G) * _C, _MC)],
                so[b])

        def wait_out(b, m):
            pltpu.make_async_copy(
                rows_v.at[b], out_hbm.at[pl.ds((irow0 + m * _G) * _C, _MC)],
                so[b]).wait()

        for m in range(_LEAD):
            load_ids(m, m)
            fire_gather(m)

        lag = _NBUF - _LEAD

        def body(q, carry):
            m0 = q * _NBUF
            for b in range(_NBUF):
                m = m0 + b
                wait_gather(b)
                fire_out(b, m)
                bn = (b + _LEAD) % _NBUF
                @pl.when(m >= lag)
                def _():
                    wait_out(bn, m - lag)

                @pl.when(m + _LEAD < macros)
                def _():
                    load_ids(bn, m + _LEAD)
                    fire_gather(bn)
            return carry

        lax.fori_loop(0, quads, body, 0)
        for k in range(lag):
            m = macros - lag + k
            wait_out(m % _NBUF, m)

    return run(ids2d, table)


def kernel(input_ids, attention_mask, emb_table):
    del attention_mask
    b, l = input_ids.shape
    n = b * l
    ids2d = input_ids.astype(jnp.int32).reshape(n // _C, _C)
    out = _sc_embed(ids2d, emb_table)
    return out.reshape(b, l, _HID)


# speedup vs baseline: 1.0130x; 1.0130x over previous
import functools

import jax
import jax.numpy as jnp
from jax import lax
from jax.experimental import pallas as pl
from jax.experimental.pallas import tpu as pltpu
from jax.experimental.pallas import tpu_sc as plsc

_HID = 128
_VOCAB = 32
_NCORES = 2
_NSUB = 16
_NW = _NCORES * _NSUB
_C = 64
_NBUF = 10
_LEAD = 6


def _sc_embed(ids2d, table):
    n_rows = ids2d.shape[0] * _C
    b_per_w = n_rows // _NW
    chunks = b_per_w // _C
    quads = chunks // _NBUF
    mesh = plsc.VectorSubcoreMesh(core_axis_name="c", subcore_axis_name="s")

    @functools.partial(
        pl.kernel,
        mesh=mesh,
        out_type=jax.ShapeDtypeStruct((n_rows, _HID), jnp.float32),
        scratch_types=[
            pltpu.VMEM((_NBUF, _C), jnp.int32),
            pltpu.VMEM((_NBUF, _C, _HID), jnp.float32),
            pltpu.VMEM_SHARED((_VOCAB, _HID), jnp.float32),
        ] + [pltpu.SemaphoreType.DMA] * (2 * _NBUF),
    )
    def run(ids_hbm, table_hbm, out_hbm, idx_v, rows_v, table_s, *sems):
        sg = sems[:_NBUF]
        so = sems[_NBUF:]
        wid = lax.axis_index("s") * _NCORES + lax.axis_index("c")
        row0 = wid * b_per_w
        irow0 = row0 // _C

        @pl.when(lax.axis_index("s") == 0)
        def _():
            pltpu.sync_copy(table_hbm, table_s)

        plsc.subcore_barrier()

        def load_ids(b, chunk):
            pltpu.sync_copy(ids_hbm.at[irow0 + chunk], idx_v.at[b])

        def fire_gather(b):
            pltpu.async_copy(table_s.at[idx_v.at[b]], rows_v.at[b], sg[b])

        def wait_gather(b):
            pltpu.make_async_copy(
                table_s.at[idx_v.at[b]], rows_v.at[b], sg[b]).wait()

        def fire_out(b, chunk):
            pltpu.async_copy(
                rows_v.at[b], out_hbm.at[pl.ds((irow0 + chunk) * _C, _C)],
                so[b])

        def wait_out(b, chunk):
            pltpu.make_async_copy(
                rows_v.at[b], out_hbm.at[pl.ds((irow0 + chunk) * _C, _C)],
                so[b]).wait()

        for c in range(_LEAD):
            load_ids(c, c)
            fire_gather(c)

        lag = _NBUF - _LEAD

        def body(q, carry):
            c0 = q * _NBUF
            for b in range(_NBUF):
                c = c0 + b
                wait_gather(b)
                fire_out(b, c)
                bn = (b + _LEAD) % _NBUF
                @pl.when(c >= lag)
                def _():
                    wait_out(bn, c - lag)

                @pl.when(c + _LEAD < chunks)
                def _():
                    load_ids(bn, c + _LEAD)
                    fire_gather(bn)
            return carry

        lax.fori_loop(0, quads, body, 0)
        for k in range(lag):
            c = chunks - lag + k
            wait_out(c % _NBUF, c)

    return run(ids2d, table)


def kernel(input_ids, attention_mask, emb_table):
    del attention_mask
    b, l = input_ids.shape
    n = b * l
    ids2d = input_ids.astype(jnp.int32).reshape(n // _C, _C)
    out = _sc_embed(ids2d, emb_table)
    return out.reshape(b, l, _HID)


# speedup vs baseline: 1.5321x; 1.5124x over previous
import functools

import jax
import jax.numpy as jnp
from jax import lax
from jax.experimental import pallas as pl
from jax.experimental.pallas import tpu as pltpu
from jax.experimental.pallas import tpu_sc as plsc

_HID = 128
_VOCAB = 32
_NCORES = 2
_NSUB = 16
_NW = _NCORES * _NSUB
_C = 128
_NBUF = 5
_LEAD = 3


def _sc_embed(ids2d, table):
    n_rows = ids2d.shape[0] * _C
    b_per_w = n_rows // _NW
    chunks = b_per_w // _C
    quads = chunks // _NBUF
    mesh = plsc.VectorSubcoreMesh(core_axis_name="c", subcore_axis_name="s")

    @functools.partial(
        pl.kernel,
        mesh=mesh,
        out_type=jax.ShapeDtypeStruct((n_rows, _HID), jnp.float32),
        scratch_types=[
            pltpu.VMEM((_NBUF, _C), jnp.int32),
            pltpu.VMEM((_NBUF, _C, _HID), jnp.float32),
            pltpu.VMEM_SHARED((_VOCAB, _HID), jnp.float32),
        ] + [pltpu.SemaphoreType.DMA] * (3 * _NBUF),
    )
    def run(ids_hbm, table_hbm, out_hbm, idx_v, rows_v, table_s, *sems):
        sg = sems[:_NBUF]
        so = sems[_NBUF:2 * _NBUF]
        si = sems[2 * _NBUF:]
        wid = lax.axis_index("s") * _NCORES + lax.axis_index("c")
        row0 = wid * b_per_w
        irow0 = row0 // _C

        @pl.when(lax.axis_index("s") == 0)
        def _():
            pltpu.sync_copy(table_hbm, table_s)

        plsc.subcore_barrier()

        def ids_copy(b, chunk):
            return pltpu.make_async_copy(
                ids_hbm.at[irow0 + chunk], idx_v.at[b], si[b])

        def fire_gather(b):
            pltpu.async_copy(table_s.at[idx_v.at[b]], rows_v.at[b], sg[b])

        def wait_gather(b):
            pltpu.make_async_copy(
                table_s.at[idx_v.at[b]], rows_v.at[b], sg[b]).wait()

        def fire_out(b, chunk):
            pltpu.async_copy(
                rows_v.at[b], out_hbm.at[pl.ds((irow0 + chunk) * _C, _C)],
                so[b])

        def wait_out(b, chunk):
            pltpu.make_async_copy(
                rows_v.at[b], out_hbm.at[pl.ds((irow0 + chunk) * _C, _C)],
                so[b]).wait()

        for c in range(_LEAD):
            ids_copy(c, c).start()
        for c in range(_LEAD):
            ids_copy(c, c).wait()
            fire_gather(c)
        ids_copy(_LEAD % _NBUF, _LEAD).start()

        lag = _NBUF - _LEAD

        def body(q, carry):
            c0 = q * _NBUF
            for b in range(_NBUF):
                c = c0 + b
                wait_gather(b)
                fire_out(b, c)
                bn = (b + _LEAD) % _NBUF
                @pl.when(c >= lag)
                def _():
                    wait_out(bn, c - lag)

                @pl.when(c + _LEAD < chunks)
                def _():
                    ids_copy(bn, c + _LEAD).wait()
                    fire_gather(bn)
                bn2 = (b + _LEAD + 1) % _NBUF
                @pl.when(c + 1 + _LEAD < chunks)
                def _():
                    ids_copy(bn2, c + 1 + _LEAD).start()
            return carry

        lax.fori_loop(0, quads, body, 0)
        for k in range(lag):
            c = chunks - lag + k
            wait_out(c % _NBUF, c)

    return run(ids2d, table)


def kernel(input_ids, attention_mask, emb_table):
    del attention_mask
    b, l = input_ids.shape
    n = b * l
    ids2d = input_ids.astype(jnp.int32).reshape(n // _C, _C)
    out = _sc_embed(ids2d, emb_table)
    return out.reshape(b, l, _HID)
